# Initial kernel scaffold; baseline (speedup 1.0000x reference)
import jax
import jax.numpy as jnp
from jax.experimental import pallas as pl


def kernel(src_node_features, dst_node_features, edge_features, src_indices, dst_indices, W1, b1, W2, b2, gamma, beta):
    raise NotImplementedError("write your pallas kernel here")


# speedup vs baseline: 2.6733x; 2.6733x over previous
import functools

import jax
import jax.numpy as jnp
from jax import lax
from jax.experimental import pallas as pl
from jax.experimental.pallas import tpu as pltpu
from jax.experimental.pallas import tpu_sc as plsc

N = 10000
E = 320000
D = 128
DE = 16
H = 128

_info = plsc.get_sparse_core_info()
_NC = _info.num_cores
_NS = _info.num_subcores
_NW = _NC * _NS
_EPW = E // _NW
_C = 400
_NCH = _EPW // _C


def _proj_body(src_ref, dst_ref, ws_ref, wd_ref, ps_ref, pd_ref):
    ps_ref[...] = jnp.dot(src_ref[...], ws_ref[...],
                          preferred_element_type=jnp.float32)
    pd_ref[...] = jnp.dot(dst_ref[...], wd_ref[...],
                          preferred_element_type=jnp.float32)


def _project(src, dst, ws, wd):
    bn = 1000
    grid = N // bn
    return pl.pallas_call(
        _proj_body,
        grid=(grid,),
        in_specs=[
            pl.BlockSpec((bn, D), lambda i: (i, 0)),
            pl.BlockSpec((bn, D), lambda i: (i, 0)),
            pl.BlockSpec((D, H), lambda i: (0, 0)),
            pl.BlockSpec((D, H), lambda i: (0, 0)),
        ],
        out_specs=[
            pl.BlockSpec((bn, H), lambda i: (i, 0)),
            pl.BlockSpec((bn, H), lambda i: (i, 0)),
        ],
        out_shape=[
            jax.ShapeDtypeStruct((N, H), jnp.float32),
            jax.ShapeDtypeStruct((N, H), jnp.float32),
        ],
    )(src, dst, ws, wd)


@functools.partial(
    pl.kernel,
    out_type=jax.ShapeDtypeStruct((E, H), jnp.float32),
    mesh=plsc.VectorSubcoreMesh(core_axis_name="c", subcore_axis_name="s"),
    scratch_types=[
        pltpu.VMEM((_C,), jnp.int32),
        pltpu.VMEM((_C,), jnp.int32),
        pltpu.VMEM((_C, H), jnp.float32),
        pltpu.VMEM((_C, H), jnp.float32),
        pltpu.SemaphoreType.DMA,
        pltpu.SemaphoreType.DMA,
    ],
)
def _sc_gather_sum(ps_hbm, pd_hbm, si_hbm, di_hbm, out_hbm,
                   idx_s, idx_d, buf_a, buf_b, sem_a, sem_b):
    wid = lax.axis_index("s") * _NC + lax.axis_index("c")
    base = wid * _EPW

    def chunk_body(ci, carry):
        off = base + ci * _C
        pltpu.sync_copy(si_hbm.at[pl.ds(off, _C)], idx_s)
        pltpu.sync_copy(di_hbm.at[pl.ds(off, _C)], idx_d)
        cpa = pltpu.async_copy(ps_hbm.at[idx_s], buf_a, sem_a)
        cpb = pltpu.async_copy(pd_hbm.at[idx_d], buf_b, sem_b)
        cpa.wait()
        cpb.wait()

        def row_body(r, c2):
            for j in range(H // 16):
                sl = pl.ds(j * 16, 16)
                buf_a[r, sl] = buf_a[r, sl] + buf_b[r, sl]
            return c2

        lax.fori_loop(0, _C, row_body, 0)
        pltpu.sync_copy(buf_a, out_hbm.at[pl.ds(off, _C)])
        return carry

    lax.fori_loop(0, _NCH, chunk_body, 0)


def _tail_body(pre_ref, ef_ref, we_ref, b1_ref, w2_ref, b2_ref,
               g_ref, bt_ref, out_ref):
    ef = ef_ref[...]
    x = pre_ref[...] + b1_ref[...]
    x = x + jnp.dot(ef, we_ref[...], preferred_element_type=jnp.float32)
    h = x * (1.0 / (1.0 + jnp.exp(-x)))
    y = jnp.dot(h, w2_ref[...], preferred_element_type=jnp.float32) + b2_ref[...]
    mu = jnp.mean(y, axis=-1, keepdims=True)
    var = jnp.mean((y - mu) * (y - mu), axis=-1, keepdims=True)
    yn = (y - mu) * lax.rsqrt(var + 1e-5) * g_ref[...] + bt_ref[...]
    out_ref[...] = yn + ef


def _tail(pre, ef, we, b1, w2, b2, gamma, beta):
    be = 2000
    grid = E // be
    return pl.pallas_call(
        _tail_body,
        grid=(grid,),
        in_specs=[
            pl.BlockSpec((be, H), lambda i: (i, 0)),
            pl.BlockSpec((be, DE), lambda i: (i, 0)),
            pl.BlockSpec((DE, H), lambda i: (0, 0)),
            pl.BlockSpec((1, H), lambda i: (0, 0)),
            pl.BlockSpec((H, DE), lambda i: (0, 0)),
            pl.BlockSpec((1, DE), lambda i: (0, 0)),
            pl.BlockSpec((1, DE), lambda i: (0, 0)),
            pl.BlockSpec((1, DE), lambda i: (0, 0)),
        ],
        out_specs=pl.BlockSpec((be, DE), lambda i: (i, 0)),
        out_shape=jax.ShapeDtypeStruct((E, DE), jnp.float32),
    )(pre, ef, we, b1, w2, b2, gamma, beta)


def kernel(src_node_features, dst_node_features, edge_features,
           src_indices, dst_indices, W1, b1, W2, b2, gamma, beta):
    ws = W1[:D]
    wd = W1[D:2 * D]
    we = W1[2 * D:]
    si = src_indices.astype(jnp.int32)
    di = dst_indices.astype(jnp.int32)
    ps, pd = _project(src_node_features, dst_node_features, ws, wd)
    pre = _sc_gather_sum(ps, pd, si, di)
    return _tail(pre, edge_features, we,
                 b1.reshape(1, H), W2, b2.reshape(1, DE),
                 gamma.reshape(1, DE), beta.reshape(1, DE))


# speedup vs baseline: 3.0472x; 1.1399x over previous
import functools

import jax
import jax.numpy as jnp
from jax import lax
from jax.experimental import pallas as pl
from jax.experimental.pallas import tpu as pltpu
from jax.experimental.pallas import tpu_sc as plsc

N = 10000
E = 320000
D = 128
DE = 16
H = 128

_info = plsc.get_sparse_core_info()
_NC = _info.num_cores
_NS = _info.num_subcores
_NW = _NC * _NS
_EPW = E // _NW
_C = 200
_NCH = _EPW // _C


def _proj_body(src_ref, dst_ref, ws_ref, wd_ref, ps_ref, pd_ref):
    ps_ref[...] = jnp.dot(src_ref[...], ws_ref[...],
                          preferred_element_type=jnp.float32)
    pd_ref[...] = jnp.dot(dst_ref[...], wd_ref[...],
                          preferred_element_type=jnp.float32)


def _project(src, dst, ws, wd):
    bn = 1000
    grid = N // bn
    return pl.pallas_call(
        _proj_body,
        grid=(grid,),
        in_specs=[
            pl.BlockSpec((bn, D), lambda i: (i, 0)),
            pl.BlockSpec((bn, D), lambda i: (i, 0)),
            pl.BlockSpec((D, H), lambda i: (0, 0)),
            pl.BlockSpec((D, H), lambda i: (0, 0)),
        ],
        out_specs=[
            pl.BlockSpec((bn, H), lambda i: (i, 0)),
            pl.BlockSpec((bn, H), lambda i: (i, 0)),
        ],
        out_shape=[
            jax.ShapeDtypeStruct((N, H), jnp.float32),
            jax.ShapeDtypeStruct((N, H), jnp.float32),
        ],
    )(src, dst, ws, wd)


@functools.partial(
    pl.kernel,
    out_type=jax.ShapeDtypeStruct((E, H), jnp.float32),
    mesh=plsc.VectorSubcoreMesh(core_axis_name="c", subcore_axis_name="s"),
    scratch_types=[
        pltpu.VMEM((_EPW,), jnp.int32),
        pltpu.VMEM((_EPW,), jnp.int32),
        pltpu.VMEM((_C, H), jnp.float32),
        pltpu.VMEM((_C, H), jnp.float32),
        pltpu.VMEM((_C, H), jnp.float32),
        pltpu.VMEM((_C, H), jnp.float32),
        pltpu.SemaphoreType.DMA,
        pltpu.SemaphoreType.DMA,
        pltpu.SemaphoreType.DMA,
        pltpu.SemaphoreType.DMA,
    ],
)
def _sc_gather_sum(ps_hbm, pd_hbm, si_hbm, di_hbm, out_hbm,
                   idx_s, idx_d, buf_a0, buf_b0, buf_a1, buf_b1,
                   sem_g0, sem_g1, sem_o0, sem_o1):
    wid = lax.axis_index("s") * _NC + lax.axis_index("c")
    base = wid * _EPW
    bufs_a = (buf_a0, buf_a1)
    bufs_b = (buf_b0, buf_b1)
    sems_g = (sem_g0, sem_g1)
    sems_o = (sem_o0, sem_o1)

    pltpu.sync_copy(si_hbm.at[pl.ds(base, _EPW)], idx_s)
    pltpu.sync_copy(di_hbm.at[pl.ds(base, _EPW)], idx_d)

    def fire(ci, b):
        loc = ci * _C
        pltpu.async_copy(ps_hbm.at[idx_s.at[pl.ds(loc, _C)]],
                         bufs_a[b], sems_g[b])
        pltpu.async_copy(pd_hbm.at[idx_d.at[pl.ds(loc, _C)]],
                         bufs_b[b], sems_g[b])

    def drain_gather(b):
        pltpu.make_async_copy(ps_hbm.at[pl.ds(0, _C)], bufs_a[b],
                              sems_g[b]).wait()
        pltpu.make_async_copy(ps_hbm.at[pl.ds(0, _C)], bufs_b[b],
                              sems_g[b]).wait()

    def add_rows(b):
        ba, bb = bufs_a[b], bufs_b[b]

        def row_body(r, c2):
            for rr in range(2):
                for j in range(H // 16):
                    sl = pl.ds(j * 16, 16)
                    ba[2 * r + rr, sl] = ba[2 * r + rr, sl] + bb[2 * r + rr, sl]
            return c2

        lax.fori_loop(0, _C // 2, row_body, 0)

    def start_out(ci, b):
        pltpu.async_copy(bufs_a[b], out_hbm.at[pl.ds(base + ci * _C, _C)],
                         sems_o[b])

    def drain_out(b):
        pltpu.make_async_copy(ps_hbm.at[pl.ds(0, _C)], bufs_a[b],
                              sems_o[b]).wait()

    fire(0, 0)

    def pair_body(g, carry):
        ci0 = 2 * g

        @pl.when(g > 0)
        def _():
            drain_out(1)
        fire(ci0 + 1, 1)
        drain_gather(0)
        add_rows(0)
        start_out(ci0, 0)

        @pl.when(g < _NCH // 2 - 1)
        def _():
            drain_out(0)
            fire(ci0 + 2, 0)
        drain_gather(1)
        add_rows(1)
        start_out(ci0 + 1, 1)
        return carry

    lax.fori_loop(0, _NCH // 2, pair_body, 0)
    drain_out(0)
    drain_out(1)


def _tail_body(pre_ref, ef_ref, we_ref, b1_ref, w2_ref, b2_ref,
               g_ref, bt_ref, out_ref):
    ef = ef_ref[...]
    x = pre_ref[...] + b1_ref[...]
    x = x + jnp.dot(ef, we_ref[...], preferred_element_type=jnp.float32)
    h = x * (1.0 / (1.0 + jnp.exp(-x)))
    y = jnp.dot(h, w2_ref[...], preferred_element_type=jnp.float32) + b2_ref[...]
    mu = jnp.mean(y, axis=-1, keepdims=True)
    var = jnp.mean((y - mu) * (y - mu), axis=-1, keepdims=True)
    yn = (y - mu) * lax.rsqrt(var + 1e-5) * g_ref[...] + bt_ref[...]
    out_ref[...] = yn + ef


def _tail(pre, ef, we, b1, w2, b2, gamma, beta):
    be = 2000
    grid = E // be
    return pl.pallas_call(
        _tail_body,
        grid=(grid,),
        in_specs=[
            pl.BlockSpec((be, H), lambda i: (i, 0)),
            pl.BlockSpec((be, DE), lambda i: (i, 0)),
            pl.BlockSpec((DE, H), lambda i: (0, 0)),
            pl.BlockSpec((1, H), lambda i: (0, 0)),
            pl.BlockSpec((H, DE), lambda i: (0, 0)),
            pl.BlockSpec((1, DE), lambda i: (0, 0)),
            pl.BlockSpec((1, DE), lambda i: (0, 0)),
            pl.BlockSpec((1, DE), lambda i: (0, 0)),
        ],
        out_specs=pl.BlockSpec((be, DE), lambda i: (i, 0)),
        out_shape=jax.ShapeDtypeStruct((E, DE), jnp.float32),
    )(pre, ef, we, b1, w2, b2, gamma, beta)


def kernel(src_node_features, dst_node_features, edge_features,
           src_indices, dst_indices, W1, b1, W2, b2, gamma, beta):
    ws = W1[:D]
    wd = W1[D:2 * D]
    we = W1[2 * D:]
    si = src_indices.astype(jnp.int32)
    di = dst_indices.astype(jnp.int32)
    ps, pd = _project(src_node_features, dst_node_features, ws, wd)
    pre = _sc_gather_sum(ps, pd, si, di)
    return _tail(pre, edge_features, we,
                 b1.reshape(1, H), W2, b2.reshape(1, DE),
                 gamma.reshape(1, DE), beta.reshape(1, DE))
